# Initial kernel scaffold; baseline (speedup 1.0000x reference)
import jax
import jax.numpy as jnp
from jax.experimental import pallas as pl


def kernel(x, table):
    raise NotImplementedError("write your pallas kernel here")


# speedup vs baseline: 1.4498x; 1.4498x over previous
import functools

import jax
import jax.numpy as jnp
from jax import lax
from jax.experimental import pallas as pl
from jax.experimental.pallas import tpu as pltpu
from jax.experimental.pallas import tpu_sc as plsc

D_MODEL = 1024
BATCH = 4
SEQ = 2048
B = BATCH * SEQ
NC, NS, L = 2, 16, 16
NW = NC * NS
BPW = B // NW
CH = 32
NCHUNK = BPW // CH
SCALE = 32.0

_mesh = plsc.VectorSubcoreMesh(core_axis_name="c", subcore_axis_name="s")


@functools.partial(
    pl.kernel,
    mesh=_mesh,
    out_type=jax.ShapeDtypeStruct((B, D_MODEL), jnp.float32),
    scratch_types=[
        pltpu.VMEM((BPW,), jnp.int32),
        pltpu.VMEM((CH, D_MODEL), jnp.float32),
        pltpu.VMEM((CH, D_MODEL), jnp.float32),
        pltpu.SemaphoreType.DMA,
        pltpu.SemaphoreType.DMA,
        pltpu.SemaphoreType.DMA,
        pltpu.SemaphoreType.DMA,
    ],
)
def _emb_lookup(x_hbm, table_hbm, out_hbm, idx_v, rows0, rows1,
                g0, g1, o0, o1):
    wid = lax.axis_index("s") * NC + lax.axis_index("c")
    base = wid * BPW
    pltpu.sync_copy(x_hbm.at[pl.ds(base, BPW)], idx_v)

    bufs = (rows0, rows1)
    gsems = (g0, g1)
    osems = (o0, o1)
    gathers = [None, None]
    outs = [None, None]

    def _scale_buf(buf):
        def srow(r, _):
            def scol(j, _):
                buf[r, pl.ds(j * L, L)] = buf[r, pl.ds(j * L, L)] * SCALE
                return 0
            return lax.fori_loop(0, D_MODEL // L, scol, 0, unroll=8)
        lax.fori_loop(0, CH, srow, 0)

    gathers[0] = pltpu.async_copy(
        table_hbm.at[idx_v.at[pl.ds(0, CH)]], bufs[0], gsems[0])

    for c in range(NCHUNK):
        b = c % 2
        nb = (c + 1) % 2
        if c + 1 < NCHUNK:
            if outs[nb] is not None:
                outs[nb].wait()
                outs[nb] = None
            gathers[nb] = pltpu.async_copy(
                table_hbm.at[idx_v.at[pl.ds((c + 1) * CH, CH)]],
                bufs[nb], gsems[nb])
        gathers[b].wait()
        _scale_buf(bufs[b])
        outs[b] = pltpu.async_copy(
            bufs[b], out_hbm.at[pl.ds(base + c * CH, CH)], osems[b])

    for b in range(2):
        if outs[b] is not None:
            outs[b].wait()


def kernel(x, table):
    out = _emb_lookup(x.reshape(B), table)
    return out.reshape(BATCH, SEQ, D_MODEL)
